# baseline (speedup 1.0000x reference)
import functools

import jax
import jax.numpy as jnp
from jax import lax
from jax.experimental import pallas as pl
from jax.experimental.pallas import tpu as pltpu
from jax.experimental.pallas import tpu_sc as plsc

_GRID = 16.0
_NUM_NEG = 16
_SOS_NEG = 8
_MARGIN = 1.0
_BIG = 1e30


def _floor16(x):
    t = x.astype(jnp.int32).astype(jnp.float32)
    return jnp.where(t > x, t - 1.0, t)


def _bilinear_weights(bx, by):
    x0 = _floor16(bx)
    y0 = _floor16(by)
    fx = bx - x0
    fy = by - y0
    return ((1.0 - fy) * (1.0 - fx), (1.0 - fy) * fx,
            fy * (1.0 - fx), fy * fx)


def _sc_sample_kernel(table_hbm, xs_hbm, ys_hbm, base_hbm, xrep_hbm, yrep_hbm,
                      out_hbm, xv, yv, basev, xrv, yrv, idxv, rows, outv, sem):
    wid = lax.axis_index("s") * 2 + lax.axis_index("c")
    start = wid * 32
    pltpu.sync_copy(xs_hbm.at[pl.ds(start, 32)], xv)
    pltpu.sync_copy(ys_hbm.at[pl.ds(start, 32)], yv)
    pltpu.sync_copy(base_hbm.at[pl.ds(start, 32)], basev)
    pltpu.sync_copy(xrep_hbm.at[pl.ds(start, 32)], xrv)
    pltpu.sync_copy(yrep_hbm.at[pl.ds(start, 32)], yrv)
    for g in range(2):
        x = xv[pl.ds(g * 16, 16)]
        y = yv[pl.ds(g * 16, 16)]
        bofs = basev[pl.ds(g * 16, 16)]
        bx = x * (1.0 / _GRID) - 0.5
        by = y * (1.0 / _GRID) - 0.5
        x0 = _floor16(bx)
        y0 = _floor16(by)
        x0c = jnp.clip(x0, 0.0, 31.0)
        x1c = jnp.clip(x0 + 1.0, 0.0, 31.0)
        y0c = jnp.clip(y0, 0.0, 31.0)
        y1c = jnp.clip(y0 + 1.0, 0.0, 31.0)
        cids = (y0c * 32.0 + x0c, y0c * 32.0 + x1c,
                y1c * 32.0 + x0c, y1c * 32.0 + x1c)
        for t, cid in enumerate(cids):
            idxv[pl.ds(t * 32 + g * 16, 16)] = bofs + cid.astype(jnp.int32)
    pltpu.async_copy(table_hbm.at[idxv], rows, sem).wait()

    def body(p, carry):
        bx = xrv[p, :] * (1.0 / _GRID) - 0.5
        by = yrv[p, :] * (1.0 / _GRID) - 0.5
        w00, w01, w10, w11 = _bilinear_weights(bx, by)
        for j in range(16):
            s = pl.ds(j * 16, 16)
            acc = (w00 * rows[p, s] + w01 * rows[p + 32, s]
                   + w10 * rows[p + 64, s] + w11 * rows[p + 96, s])
            outv[p, s] = acc
        return carry

    lax.fori_loop(0, 32, body, 0)
    pltpu.sync_copy(outv, out_hbm.at[pl.ds(start, 32)])


def _sc_sample(table, xs, ys, base):
    bn, c = xs.shape[0], table.shape[1]
    xrep = jnp.broadcast_to(xs[:, None], (bn, 16))
    yrep = jnp.broadcast_to(ys[:, None], (bn, 16))
    mesh = plsc.VectorSubcoreMesh(core_axis_name="c", subcore_axis_name="s")
    f = functools.partial(
        pl.kernel, mesh=mesh,
        out_type=jax.ShapeDtypeStruct((bn, c), jnp.float32),
        scratch_types=[
            pltpu.VMEM((32,), jnp.float32),
            pltpu.VMEM((32,), jnp.float32),
            pltpu.VMEM((32,), jnp.int32),
            pltpu.VMEM((32, 16), jnp.float32),
            pltpu.VMEM((32, 16), jnp.float32),
            pltpu.VMEM((128,), jnp.int32),
            pltpu.VMEM((128, c), jnp.float32),
            pltpu.VMEM((32, c), jnp.float32),
            pltpu.SemaphoreType.DMA,
        ],
    )(_sc_sample_kernel)
    return f(table, xs, ys, base, xrep, yrep)


def _nearest4(x, y):
    n = x.shape[0]
    jx = jnp.clip(jnp.floor(x * (1.0 / _GRID)), 0.0, 31.0)
    jy = jnp.clip(jnp.floor(y * (1.0 / _GRID)), 0.0, 31.0)
    c0 = jnp.clip(jx - 2.0, 0.0, 27.0)
    r0 = jnp.clip(jy - 2.0, 0.0, 27.0)
    lane = jax.lax.broadcasted_iota(jnp.int32, (n, 25), 1).astype(jnp.float32)
    dcol = lane - 5.0 * jnp.floor(lane * 0.2)
    drow = jnp.floor(lane * 0.2)
    cols = c0 + dcol
    rows = r0 + drow
    cx = cols * _GRID + 8.0
    cy = rows * _GRID + 8.0
    dx = x - cx
    dy = y - cy
    d2 = dx * dx + dy * dy
    idx = rows * 32.0 + cols
    ids = []
    for _ in range(4):
        m = jnp.min(d2, axis=1, keepdims=True)
        sel = jnp.min(jnp.where(d2 == m, idx, jnp.float32(4096.0)),
                      axis=1, keepdims=True)
        ids.append(sel)
        d2 = jnp.where(idx == sel, jnp.float32(_BIG), d2)
    return ids


def _onehot4(ids, lane_hw):
    acc = None
    for s in ids:
        plane = (lane_hw == s).astype(jnp.float32)
        acc = plane if acc is None else acc + plane
    return acc


def _mine_kernel(homo_ref, kp1_ref, wkp1_ref, kd_ref, d_ref, o1_ref, o2_ref):
    i = pl.program_id(0)
    n = kd_ref.shape[1]
    hw = d_ref.shape[2]

    kd = kd_ref[0]
    D = d_ref[0]

    kx = kp1_ref[0, :, 0:1]
    ky = kp1_ref[0, :, 1:2]
    wx = wkp1_ref[0, :, 0:1]
    wy = wkp1_ref[0, :, 1:2]

    lane_hw = jax.lax.broadcasted_iota(jnp.int32, (n, hw), 1).astype(jnp.float32)

    kids = _nearest4(kx, ky)
    wids = _nearest4(wx, wy)
    K = _onehot4(kids, lane_hw)
    W = _onehot4(wids, lane_hw)

    h00 = homo_ref[i, 0]
    h01 = homo_ref[i, 1]
    h02 = homo_ref[i, 2]
    h10 = homo_ref[i, 3]
    h11 = homo_ref[i, 4]
    h12 = homo_ref[i, 5]
    h20 = homo_ref[i, 6]
    h21 = homo_ref[i, 7]
    h22 = homo_ref[i, 8]
    N = None
    for p in range(4):
        cidf = kids[p]
        row = jnp.floor(cidf * (1.0 / 32.0))
        col = cidf - 32.0 * row
        cx = col * _GRID + 8.0
        cy = row * _GRID + 8.0
        wz = h20 * cx + h21 * cy + h22
        px = (h00 * cx + h01 * cy + h02) / (wz + 1e-8)
        py = (h10 * cx + h11 * cy + h12) / (wz + 1e-8)
        cnt = _onehot4(_nearest4(px, py), lane_hw)
        N = cnt if N is None else N + cnt

    S = jax.lax.dot_general(kd, D, (((1,), (0,)), ((), ())),
                            preferred_element_type=jnp.float32)
    X = 2.0 - 2.0 * S + 5.0 * N
    hwf = jnp.float32(hw)
    lane_o = jax.lax.broadcasted_iota(jnp.int32, (n, 128), 1)
    o1 = jnp.zeros((n, 128), jnp.float32)
    for k in range(_NUM_NEG):
        m = jnp.min(X, axis=1, keepdims=True)
        sel = jnp.min(jnp.where(X == m, lane_hw, hwf), axis=1, keepdims=True)
        X = jnp.where(lane_hw == sel, jnp.float32(_BIG), X)
        o1 = jnp.where(lane_o == k, m, o1)

    nt = (((1,), (1,)), ((), ()))
    Km = jax.lax.dot_general(K, K, nt, preferred_element_type=jnp.float32)
    Wm = jax.lax.dot_general(N, W, nt, preferred_element_type=jnp.float32)
    kraw = 2.0 - 2.0 * jax.lax.dot_general(kd, kd, nt,
                                           preferred_element_type=jnp.float32)
    Xa = kraw + 5.0 * Km
    lane_n = jax.lax.broadcasted_iota(jnp.int32, (n, n), 1).astype(jnp.float32)
    nf = jnp.float32(n)
    for k in range(_SOS_NEG):
        ma = jnp.min(Xa, axis=1, keepdims=True)
        sa = jnp.min(jnp.where(Xa == ma, lane_n, nf), axis=1, keepdims=True)
        ea = lane_n == sa
        va = jnp.sum(jnp.where(ea, kraw, 0.0), axis=1, keepdims=True)
        Xa = jnp.where(ea, jnp.float32(_BIG), Xa)
        o1 = jnp.where(lane_o == _NUM_NEG + k, va, o1)

    o1_ref[0] = o1
    o2_ref[0] = Wm


def _combine_kernel(kd_ref, wdu_ref, o1_ref, o2_ref, out_ref):
    n = kd_ref.shape[1]
    kd = kd_ref[0]

    wd = wdu_ref[0]
    wd = wd * jax.lax.rsqrt(jnp.sum(wd * wd, axis=1, keepdims=True) + 1e-12)

    pos = 2.0 - 2.0 * jnp.sum(kd * wd, axis=1, keepdims=True)

    o1 = o1_ref[0]
    m16 = o1[:, 0:_NUM_NEG]
    t = jnp.maximum(pos - m16 + _MARGIN, 0.0)
    fos_sum = jnp.sum(t * t)

    nt = (((1,), (1,)), ((), ()))
    wraw = 2.0 - 2.0 * jax.lax.dot_general(wd, wd, nt,
                                           preferred_element_type=jnp.float32)
    Xb = wraw + 5.0 * o2_ref[0]
    lane_n = jax.lax.broadcasted_iota(jnp.int32, (n, n), 1).astype(jnp.float32)
    nf = jnp.float32(n)
    sacc = jnp.zeros((n, 1), jnp.float32)
    for k in range(_SOS_NEG):
        mb = jnp.min(Xb, axis=1, keepdims=True)
        sb = jnp.min(jnp.where(Xb == mb, lane_n, nf), axis=1, keepdims=True)
        eb = lane_n == sb
        vb = jnp.sum(jnp.where(eb, wraw, 0.0), axis=1, keepdims=True)
        Xb = jnp.where(eb, jnp.float32(_BIG), Xb)
        d = o1[:, _NUM_NEG + k:_NUM_NEG + k + 1] - vb
        sacc = sacc + d * d
    sos_sum = jnp.sum(jnp.sqrt(sacc + 1e-12))

    lane_o = jax.lax.broadcasted_iota(jnp.int32, (1, 128), 1)
    out_ref[0] = jnp.where(lane_o == 0, fos_sum,
                           jnp.where(lane_o == 1, sos_sum, 0.0))


@jax.jit
def kernel(kp1, w_kp1, kp1_desc, desc2, homo12):
    b, n, c = kp1_desc.shape
    h, w = desc2.shape[2], desc2.shape[3]
    hw = h * w
    D = desc2.reshape(b, c, hw)
    homo_flat = homo12.reshape(b, 9)

    table = jnp.transpose(D, (0, 2, 1)).reshape(b * hw, c)
    xs = w_kp1[..., 0].reshape(b * n)
    ys = w_kp1[..., 1].reshape(b * n)
    base = jnp.repeat(jnp.arange(b, dtype=jnp.int32) * hw, n)
    wdu = _sc_sample(table, xs, ys, base).reshape(b, n, c)

    grid_spec = pltpu.PrefetchScalarGridSpec(
        num_scalar_prefetch=1,
        grid=(b,),
        in_specs=[
            pl.BlockSpec((1, n, 2), lambda i, s: (i, 0, 0)),
            pl.BlockSpec((1, n, 2), lambda i, s: (i, 0, 0)),
            pl.BlockSpec((1, n, c), lambda i, s: (i, 0, 0)),
            pl.BlockSpec((1, c, hw), lambda i, s: (i, 0, 0)),
        ],
        out_specs=[
            pl.BlockSpec((1, n, 128), lambda i, s: (i, 0, 0)),
            pl.BlockSpec((1, n, n), lambda i, s: (i, 0, 0)),
        ],
    )
    o1, o2 = pl.pallas_call(
        _mine_kernel,
        grid_spec=grid_spec,
        out_shape=[
            jax.ShapeDtypeStruct((b, n, 128), jnp.float32),
            jax.ShapeDtypeStruct((b, n, n), jnp.float32),
        ],
    )(homo_flat, kp1, w_kp1, kp1_desc, D)

    part = pl.pallas_call(
        _combine_kernel,
        grid=(b,),
        in_specs=[
            pl.BlockSpec((1, n, c), lambda i: (i, 0, 0)),
            pl.BlockSpec((1, n, c), lambda i: (i, 0, 0)),
            pl.BlockSpec((1, n, 128), lambda i: (i, 0, 0)),
            pl.BlockSpec((1, n, n), lambda i: (i, 0, 0)),
        ],
        out_specs=pl.BlockSpec((1, 1, 128), lambda i: (i, 0, 0)),
        out_shape=jax.ShapeDtypeStruct((b, 1, 128), jnp.float32),
    )(kp1_desc, wdu, o1, o2)
    fos = jnp.sum(part[:, 0, 0]) / (b * n * _NUM_NEG)
    sos = jnp.sum(part[:, 0, 1]) / (b * n)
    return fos + sos


# speedup vs baseline: 1.0007x; 1.0007x over previous
import functools

import jax
import jax.numpy as jnp
from jax import lax
from jax.experimental import pallas as pl
from jax.experimental.pallas import tpu as pltpu
from jax.experimental.pallas import tpu_sc as plsc

_GRID = 16.0
_NUM_NEG = 16
_SOS_NEG = 8
_MARGIN = 1.0
_BIG = 1e30


def _floor16(x):
    t = x.astype(jnp.int32).astype(jnp.float32)
    return jnp.where(t > x, t - 1.0, t)


def _bilinear_weights(bx, by):
    x0 = _floor16(bx)
    y0 = _floor16(by)
    fx = bx - x0
    fy = by - y0
    return ((1.0 - fy) * (1.0 - fx), (1.0 - fy) * fx,
            fy * (1.0 - fx), fy * fx)


def _sc_sample_kernel(table_hbm, xs_hbm, ys_hbm, base_hbm, xrep_hbm, yrep_hbm,
                      out_hbm, xv, yv, basev, xrv, yrv, idxv, rows, outv, sem):
    wid = lax.axis_index("s") * 2 + lax.axis_index("c")
    start = wid * 32
    pltpu.sync_copy(xs_hbm.at[pl.ds(start, 32)], xv)
    pltpu.sync_copy(ys_hbm.at[pl.ds(start, 32)], yv)
    pltpu.sync_copy(base_hbm.at[pl.ds(start, 32)], basev)
    pltpu.sync_copy(xrep_hbm.at[pl.ds(start, 32)], xrv)
    pltpu.sync_copy(yrep_hbm.at[pl.ds(start, 32)], yrv)
    for g in range(2):
        x = xv[pl.ds(g * 16, 16)]
        y = yv[pl.ds(g * 16, 16)]
        bofs = basev[pl.ds(g * 16, 16)]
        bx = x * (1.0 / _GRID) - 0.5
        by = y * (1.0 / _GRID) - 0.5
        x0 = _floor16(bx)
        y0 = _floor16(by)
        x0c = jnp.clip(x0, 0.0, 31.0)
        x1c = jnp.clip(x0 + 1.0, 0.0, 31.0)
        y0c = jnp.clip(y0, 0.0, 31.0)
        y1c = jnp.clip(y0 + 1.0, 0.0, 31.0)
        cids = (y0c * 32.0 + x0c, y0c * 32.0 + x1c,
                y1c * 32.0 + x0c, y1c * 32.0 + x1c)
        for t, cid in enumerate(cids):
            idxv[pl.ds(t * 32 + g * 16, 16)] = bofs + cid.astype(jnp.int32)
    pltpu.async_copy(table_hbm.at[idxv], rows, sem).wait()

    def body(p, carry):
        bx = xrv[p, :] * (1.0 / _GRID) - 0.5
        by = yrv[p, :] * (1.0 / _GRID) - 0.5
        w00, w01, w10, w11 = _bilinear_weights(bx, by)
        for j in range(16):
            s = pl.ds(j * 16, 16)
            acc = (w00 * rows[p, s] + w01 * rows[p + 32, s]
                   + w10 * rows[p + 64, s] + w11 * rows[p + 96, s])
            outv[p, s] = acc
        return carry

    lax.fori_loop(0, 32, body, 0)
    pltpu.sync_copy(outv, out_hbm.at[pl.ds(start, 32)])


def _sc_sample(table, xs, ys, base):
    bn, c = xs.shape[0], table.shape[1]
    xrep = jnp.broadcast_to(xs[:, None], (bn, 16))
    yrep = jnp.broadcast_to(ys[:, None], (bn, 16))
    mesh = plsc.VectorSubcoreMesh(core_axis_name="c", subcore_axis_name="s")
    f = functools.partial(
        pl.kernel, mesh=mesh,
        out_type=jax.ShapeDtypeStruct((bn, c), jnp.float32),
        scratch_types=[
            pltpu.VMEM((32,), jnp.float32),
            pltpu.VMEM((32,), jnp.float32),
            pltpu.VMEM((32,), jnp.int32),
            pltpu.VMEM((32, 16), jnp.float32),
            pltpu.VMEM((32, 16), jnp.float32),
            pltpu.VMEM((128,), jnp.int32),
            pltpu.VMEM((128, c), jnp.float32),
            pltpu.VMEM((32, c), jnp.float32),
            pltpu.SemaphoreType.DMA,
        ],
    )(_sc_sample_kernel)
    return f(table, xs, ys, base, xrep, yrep)


def _nearest4(x, y):
    n = x.shape[0]
    jx = jnp.clip(jnp.floor(x * (1.0 / _GRID)), 0.0, 31.0)
    jy = jnp.clip(jnp.floor(y * (1.0 / _GRID)), 0.0, 31.0)
    c0 = jnp.clip(jx - 2.0, 0.0, 27.0)
    r0 = jnp.clip(jy - 2.0, 0.0, 27.0)
    lane = jax.lax.broadcasted_iota(jnp.int32, (n, 25), 1).astype(jnp.float32)
    dcol = lane - 5.0 * jnp.floor(lane * 0.2)
    drow = jnp.floor(lane * 0.2)
    cols = c0 + dcol
    rows = r0 + drow
    cx = cols * _GRID + 8.0
    cy = rows * _GRID + 8.0
    dx = x - cx
    dy = y - cy
    d2 = dx * dx + dy * dy
    idx = rows * 32.0 + cols
    ids = []
    for _ in range(4):
        m = jnp.min(d2, axis=1, keepdims=True)
        sel = jnp.min(jnp.where(d2 == m, idx, jnp.float32(4096.0)),
                      axis=1, keepdims=True)
        ids.append(sel)
        d2 = jnp.where(idx == sel, jnp.float32(_BIG), d2)
    return ids


def _onehot4(ids, lane_hw):
    acc = None
    for s in ids:
        plane = (lane_hw == s).astype(jnp.float32)
        acc = plane if acc is None else acc + plane
    return acc


def _mine_kernel(homo_ref, kp1_ref, wkp1_ref, kd_ref, d_ref, o1_ref, o2_ref):
    i = pl.program_id(0)
    n = kd_ref.shape[1]
    hw = d_ref.shape[2]

    kd = kd_ref[0]
    D = d_ref[0]

    kx = kp1_ref[0, :, 0:1]
    ky = kp1_ref[0, :, 1:2]
    wx = wkp1_ref[0, :, 0:1]
    wy = wkp1_ref[0, :, 1:2]

    lane_hw = jax.lax.broadcasted_iota(jnp.int32, (n, hw), 1).astype(jnp.float32)

    kids = _nearest4(kx, ky)
    wids = _nearest4(wx, wy)
    K = _onehot4(kids, lane_hw)
    W = _onehot4(wids, lane_hw)

    h00 = homo_ref[i, 0]
    h01 = homo_ref[i, 1]
    h02 = homo_ref[i, 2]
    h10 = homo_ref[i, 3]
    h11 = homo_ref[i, 4]
    h12 = homo_ref[i, 5]
    h20 = homo_ref[i, 6]
    h21 = homo_ref[i, 7]
    h22 = homo_ref[i, 8]
    N = None
    for p in range(4):
        cidf = kids[p]
        row = jnp.floor(cidf * (1.0 / 32.0))
        col = cidf - 32.0 * row
        cx = col * _GRID + 8.0
        cy = row * _GRID + 8.0
        wz = h20 * cx + h21 * cy + h22
        px = (h00 * cx + h01 * cy + h02) / (wz + 1e-8)
        py = (h10 * cx + h11 * cy + h12) / (wz + 1e-8)
        cnt = _onehot4(_nearest4(px, py), lane_hw)
        N = cnt if N is None else N + cnt

    S = jax.lax.dot_general(kd, D, (((1,), (0,)), ((), ())),
                            preferred_element_type=jnp.float32)
    X = 2.0 - 2.0 * S + 5.0 * N
    hwf = jnp.float32(hw)
    lane_o = jax.lax.broadcasted_iota(jnp.int32, (n, 128), 1)
    o1 = jnp.zeros((n, 128), jnp.float32)
    for k in range(_NUM_NEG):
        m = jnp.min(X, axis=1, keepdims=True)
        sel = jnp.min(jnp.where(X == m, lane_hw, hwf), axis=1, keepdims=True)
        X = jnp.where(lane_hw == sel, jnp.float32(_BIG), X)
        o1 = jnp.where(lane_o == k, m, o1)

    nt = (((1,), (1,)), ((), ()))
    Km = jax.lax.dot_general(K, K, nt, preferred_element_type=jnp.float32)
    Wm = jax.lax.dot_general(N, W, nt, preferred_element_type=jnp.float32)
    kraw = 2.0 - 2.0 * jax.lax.dot_general(kd, kd, nt,
                                           preferred_element_type=jnp.float32)
    Xa = kraw + 5.0 * Km
    lane_n = jax.lax.broadcasted_iota(jnp.int32, (n, n), 1).astype(jnp.float32)
    nf = jnp.float32(n)
    for k in range(_SOS_NEG):
        ma = jnp.min(Xa, axis=1, keepdims=True)
        sa = jnp.min(jnp.where(Xa == ma, lane_n, nf), axis=1, keepdims=True)
        ea = lane_n == sa
        va = jnp.sum(jnp.where(ea, kraw, 0.0), axis=1, keepdims=True)
        Xa = jnp.where(ea, jnp.float32(_BIG), Xa)
        o1 = jnp.where(lane_o == _NUM_NEG + k, va, o1)

    o1_ref[0] = o1
    o2_ref[0] = Wm


def _combine_kernel(kd_ref, wdu_ref, o1_ref, o2_ref, out_ref):
    n = kd_ref.shape[1]
    kd = kd_ref[0]

    wd = wdu_ref[0]
    wd = wd * jax.lax.rsqrt(jnp.sum(wd * wd, axis=1, keepdims=True) + 1e-12)

    pos = 2.0 - 2.0 * jnp.sum(kd * wd, axis=1, keepdims=True)

    o1 = o1_ref[0]
    m16 = o1[:, 0:_NUM_NEG]
    t = jnp.maximum(pos - m16 + _MARGIN, 0.0)
    fos_sum = jnp.sum(t * t)

    nt = (((1,), (1,)), ((), ()))
    wraw = 2.0 - 2.0 * jax.lax.dot_general(wd, wd, nt,
                                           preferred_element_type=jnp.float32)
    Xb = wraw + 5.0 * o2_ref[0]
    lane_n = jax.lax.broadcasted_iota(jnp.int32, (n, n), 1).astype(jnp.float32)
    nf = jnp.float32(n)
    sacc = jnp.zeros((n, 1), jnp.float32)
    for k in range(_SOS_NEG):
        mb = jnp.min(Xb, axis=1, keepdims=True)
        sb = jnp.min(jnp.where(Xb == mb, lane_n, nf), axis=1, keepdims=True)
        eb = lane_n == sb
        vb = jnp.sum(jnp.where(eb, wraw, 0.0), axis=1, keepdims=True)
        Xb = jnp.where(eb, jnp.float32(_BIG), Xb)
        d = o1[:, _NUM_NEG + k:_NUM_NEG + k + 1] - vb
        sacc = sacc + d * d
    sos_sum = jnp.sum(jnp.sqrt(sacc + 1e-12))

    lane_o = jax.lax.broadcasted_iota(jnp.int32, (1, 128), 1)
    out_ref[0] = jnp.where(lane_o == 0, fos_sum,
                           jnp.where(lane_o == 1, sos_sum, 0.0))


@jax.jit
def kernel(kp1, w_kp1, kp1_desc, desc2, homo12):
    b, n, c = kp1_desc.shape
    h, w = desc2.shape[2], desc2.shape[3]
    hw = h * w
    D = desc2.reshape(b, c, hw)
    homo_flat = homo12.reshape(b, 9)

    grid_spec = pltpu.PrefetchScalarGridSpec(
        num_scalar_prefetch=1,
        grid=(b,),
        in_specs=[
            pl.BlockSpec((1, n, 2), lambda i, s: (i, 0, 0)),
            pl.BlockSpec((1, n, 2), lambda i, s: (i, 0, 0)),
            pl.BlockSpec((1, n, c), lambda i, s: (i, 0, 0)),
            pl.BlockSpec((1, c, hw), lambda i, s: (i, 0, 0)),
        ],
        out_specs=[
            pl.BlockSpec((1, n, 128), lambda i, s: (i, 0, 0)),
            pl.BlockSpec((1, n, n), lambda i, s: (i, 0, 0)),
        ],
    )
    o1, o2 = pl.pallas_call(
        _mine_kernel,
        grid_spec=grid_spec,
        out_shape=[
            jax.ShapeDtypeStruct((b, n, 128), jnp.float32),
            jax.ShapeDtypeStruct((b, n, n), jnp.float32),
        ],
    )(homo_flat, kp1, w_kp1, kp1_desc, D)

    table = jnp.transpose(D, (0, 2, 1)).reshape(b * hw, c)
    xs = w_kp1[..., 0].reshape(b * n)
    ys = w_kp1[..., 1].reshape(b * n)
    base = jnp.repeat(jnp.arange(b, dtype=jnp.int32) * hw, n)
    wdu = _sc_sample(table, xs, ys, base).reshape(b, n, c)

    part = pl.pallas_call(
        _combine_kernel,
        grid=(b,),
        in_specs=[
            pl.BlockSpec((1, n, c), lambda i: (i, 0, 0)),
            pl.BlockSpec((1, n, c), lambda i: (i, 0, 0)),
            pl.BlockSpec((1, n, 128), lambda i: (i, 0, 0)),
            pl.BlockSpec((1, n, n), lambda i: (i, 0, 0)),
        ],
        out_specs=pl.BlockSpec((1, 1, 128), lambda i: (i, 0, 0)),
        out_shape=jax.ShapeDtypeStruct((b, 1, 128), jnp.float32),
    )(kp1_desc, wdu, o1, o2)
    fos = jnp.sum(part[:, 0, 0]) / (b * n * _NUM_NEG)
    sos = jnp.sum(part[:, 0, 1]) / (b * n)
    return fos + sos


# speedup vs baseline: 1.0716x; 1.0709x over previous
import functools

import jax
import jax.numpy as jnp
from jax import lax
from jax.experimental import pallas as pl
from jax.experimental.pallas import tpu as pltpu
from jax.experimental.pallas import tpu_sc as plsc

_GRID = 16.0
_NUM_NEG = 16
_SOS_NEG = 8
_MARGIN = 1.0
_BIG = 1e30


def _floor16(x):
    t = x.astype(jnp.int32).astype(jnp.float32)
    return jnp.where(t > x, t - 1.0, t)


def _sc_sample_kernel(table_hbm, xs_hbm, ys_hbm, xyrep_hbm, out_hbm,
                      xv, yv, xyrv, idxv, rows, outv, sem):
    wid = lax.axis_index("s") * 2 + lax.axis_index("c")
    start = wid * 32
    c1 = pltpu.async_copy(xs_hbm.at[pl.ds(start, 32)], xv, sem)
    c2 = pltpu.async_copy(ys_hbm.at[pl.ds(start, 32)], yv, sem)
    c3 = pltpu.async_copy(xyrep_hbm.at[pl.ds(start, 32)], xyrv, sem)
    c1.wait()
    c2.wait()
    c3.wait()
    lanes = lax.iota(jnp.int32, 16)
    for g in range(2):
        x = xv[pl.ds(g * 16, 16)]
        y = yv[pl.ds(g * 16, 16)]
        gid = lanes + (start + g * 16)
        bofs = lax.shift_left(lax.shift_right_logical(gid, 8), 10)
        bx = x * (1.0 / _GRID) - 0.5
        by = y * (1.0 / _GRID) - 0.5
        x0 = _floor16(bx)
        y0 = _floor16(by)
        x0c = jnp.clip(x0, 0.0, 31.0)
        x1c = jnp.clip(x0 + 1.0, 0.0, 31.0)
        y0c = jnp.clip(y0, 0.0, 31.0)
        y1c = jnp.clip(y0 + 1.0, 0.0, 31.0)
        cids = (y0c * 32.0 + x0c, y0c * 32.0 + x1c,
                y1c * 32.0 + x0c, y1c * 32.0 + x1c)
        for t, cid in enumerate(cids):
            idxv[pl.ds(t * 32 + g * 16, 16)] = bofs + cid.astype(jnp.int32)
    pltpu.async_copy(table_hbm.at[idxv], rows, sem).wait()

    def body(p, carry):
        bx = xyrv[p, pl.ds(0, 16)] * (1.0 / _GRID) - 0.5
        by = xyrv[p, pl.ds(16, 16)] * (1.0 / _GRID) - 0.5
        x0 = _floor16(bx)
        y0 = _floor16(by)
        fx = bx - x0
        fy = by - y0
        w00 = (1.0 - fy) * (1.0 - fx)
        w01 = (1.0 - fy) * fx
        w10 = fy * (1.0 - fx)
        w11 = fy * fx
        for j in range(16):
            s = pl.ds(j * 16, 16)
            acc = (w00 * rows[p, s] + w01 * rows[p + 32, s]
                   + w10 * rows[p + 64, s] + w11 * rows[p + 96, s])
            outv[p, s] = acc
        return carry

    lax.fori_loop(0, 32, body, 0)
    pltpu.sync_copy(outv, out_hbm.at[pl.ds(start, 32)])


def _sc_sample(table, xs, ys):
    bn, c = xs.shape[0], table.shape[1]
    xyrep = jnp.concatenate(
        [jnp.broadcast_to(xs[:, None], (bn, 16)),
         jnp.broadcast_to(ys[:, None], (bn, 16))], axis=1)
    mesh = plsc.VectorSubcoreMesh(core_axis_name="c", subcore_axis_name="s")
    f = functools.partial(
        pl.kernel, mesh=mesh,
        out_type=jax.ShapeDtypeStruct((bn, c), jnp.float32),
        scratch_types=[
            pltpu.VMEM((32,), jnp.float32),
            pltpu.VMEM((32,), jnp.float32),
            pltpu.VMEM((32, 32), jnp.float32),
            pltpu.VMEM((128,), jnp.int32),
            pltpu.VMEM((128, c), jnp.float32),
            pltpu.VMEM((32, c), jnp.float32),
            pltpu.SemaphoreType.DMA,
        ],
    )(_sc_sample_kernel)
    return f(table, xs, ys, xyrep)


def _nearest4(x, y):
    n = x.shape[0]
    jx = jnp.clip(jnp.floor(x * (1.0 / _GRID)), 0.0, 31.0)
    jy = jnp.clip(jnp.floor(y * (1.0 / _GRID)), 0.0, 31.0)
    c0 = jnp.clip(jx - 2.0, 0.0, 27.0)
    r0 = jnp.clip(jy - 2.0, 0.0, 27.0)
    lane = jax.lax.broadcasted_iota(jnp.int32, (n, 25), 1).astype(jnp.float32)
    dcol = lane - 5.0 * jnp.floor(lane * 0.2)
    drow = jnp.floor(lane * 0.2)
    cols = c0 + dcol
    rows = r0 + drow
    cx = cols * _GRID + 8.0
    cy = rows * _GRID + 8.0
    dx = x - cx
    dy = y - cy
    d2 = dx * dx + dy * dy
    idx = rows * 32.0 + cols
    ids = []
    for _ in range(4):
        m = jnp.min(d2, axis=1, keepdims=True)
        sel = jnp.min(jnp.where(d2 == m, idx, jnp.float32(4096.0)),
                      axis=1, keepdims=True)
        ids.append(sel)
        d2 = jnp.where(idx == sel, jnp.float32(_BIG), d2)
    return ids


def _onehot4(ids, lane_hw):
    acc = None
    for s in ids:
        plane = (lane_hw == s).astype(jnp.float32)
        acc = plane if acc is None else acc + plane
    return acc


def _loss_kernel(homo_ref, kp1_ref, wkp1_ref, kd_ref, d_ref, wdu_ref, out_ref):
    i = pl.program_id(0)
    n = kd_ref.shape[1]
    hw = d_ref.shape[2]

    kd = kd_ref[0]
    D = d_ref[0]

    kx = kp1_ref[0, :, 0:1]
    ky = kp1_ref[0, :, 1:2]
    wx = wkp1_ref[0, :, 0:1]
    wy = wkp1_ref[0, :, 1:2]

    lane_hw = jax.lax.broadcasted_iota(jnp.int32, (n, hw), 1).astype(jnp.float32)

    kids = _nearest4(kx, ky)
    wids = _nearest4(wx, wy)
    K = _onehot4(kids, lane_hw)
    W = _onehot4(wids, lane_hw)

    h00 = homo_ref[i, 0]
    h01 = homo_ref[i, 1]
    h02 = homo_ref[i, 2]
    h10 = homo_ref[i, 3]
    h11 = homo_ref[i, 4]
    h12 = homo_ref[i, 5]
    h20 = homo_ref[i, 6]
    h21 = homo_ref[i, 7]
    h22 = homo_ref[i, 8]
    N = None
    for p in range(4):
        cidf = kids[p]
        row = jnp.floor(cidf * (1.0 / 32.0))
        col = cidf - 32.0 * row
        cx = col * _GRID + 8.0
        cy = row * _GRID + 8.0
        wz = h20 * cx + h21 * cy + h22
        px = (h00 * cx + h01 * cy + h02) / (wz + 1e-8)
        py = (h10 * cx + h11 * cy + h12) / (wz + 1e-8)
        cnt = _onehot4(_nearest4(px, py), lane_hw)
        N = cnt if N is None else N + cnt

    wd = wdu_ref[0]
    wd = wd * jax.lax.rsqrt(jnp.sum(wd * wd, axis=1, keepdims=True) + 1e-12)

    pos = 2.0 - 2.0 * jnp.sum(kd * wd, axis=1, keepdims=True)

    S = jax.lax.dot_general(kd, D, (((1,), (0,)), ((), ())),
                            preferred_element_type=jnp.float32)
    X = 2.0 - 2.0 * S + 5.0 * N
    fos_sum = jnp.float32(0.0)
    hwf = jnp.float32(hw)
    for _ in range(_NUM_NEG):
        m = jnp.min(X, axis=1, keepdims=True)
        sel = jnp.min(jnp.where(X == m, lane_hw, hwf), axis=1, keepdims=True)
        X = jnp.where(lane_hw == sel, jnp.float32(_BIG), X)
        t = jnp.maximum(pos - m + _MARGIN, 0.0)
        fos_sum = fos_sum + jnp.sum(t * t)

    nt = (((1,), (1,)), ((), ()))
    Km = jax.lax.dot_general(K, K, nt, preferred_element_type=jnp.float32)
    Wm = jax.lax.dot_general(N, W, nt, preferred_element_type=jnp.float32)
    kraw = 2.0 - 2.0 * jax.lax.dot_general(kd, kd, nt,
                                           preferred_element_type=jnp.float32)
    wraw = 2.0 - 2.0 * jax.lax.dot_general(wd, wd, nt,
                                           preferred_element_type=jnp.float32)
    Xa = kraw + 5.0 * Km
    Xb = wraw + 5.0 * Wm
    lane_n = jax.lax.broadcasted_iota(jnp.int32, (n, n), 1).astype(jnp.float32)
    nf = jnp.float32(n)
    sacc = jnp.zeros((n, 1), jnp.float32)
    for _ in range(_SOS_NEG):
        ma = jnp.min(Xa, axis=1, keepdims=True)
        sa = jnp.min(jnp.where(Xa == ma, lane_n, nf), axis=1, keepdims=True)
        ea = lane_n == sa
        va = jnp.sum(jnp.where(ea, kraw, 0.0), axis=1, keepdims=True)
        Xa = jnp.where(ea, jnp.float32(_BIG), Xa)
        mb = jnp.min(Xb, axis=1, keepdims=True)
        sb = jnp.min(jnp.where(Xb == mb, lane_n, nf), axis=1, keepdims=True)
        eb = lane_n == sb
        vb = jnp.sum(jnp.where(eb, wraw, 0.0), axis=1, keepdims=True)
        Xb = jnp.where(eb, jnp.float32(_BIG), Xb)
        d = va - vb
        sacc = sacc + d * d
    sos_sum = jnp.sum(jnp.sqrt(sacc + 1e-12))

    lane_o = jax.lax.broadcasted_iota(jnp.int32, (1, 128), 1)
    out_ref[0] = jnp.where(lane_o == 0, fos_sum,
                           jnp.where(lane_o == 1, sos_sum, 0.0))


@jax.jit
def kernel(kp1, w_kp1, kp1_desc, desc2, homo12):
    b, n, c = kp1_desc.shape
    h, w = desc2.shape[2], desc2.shape[3]
    hw = h * w
    D = desc2.reshape(b, c, hw)
    homo_flat = homo12.reshape(b, 9)

    table = jnp.transpose(D, (0, 2, 1)).reshape(b * hw, c)
    xs = w_kp1[..., 0].reshape(b * n)
    ys = w_kp1[..., 1].reshape(b * n)
    wdu = _sc_sample(table, xs, ys).reshape(b, n, c)

    grid_spec = pltpu.PrefetchScalarGridSpec(
        num_scalar_prefetch=1,
        grid=(b,),
        in_specs=[
            pl.BlockSpec((1, n, 2), lambda i, s: (i, 0, 0)),
            pl.BlockSpec((1, n, 2), lambda i, s: (i, 0, 0)),
            pl.BlockSpec((1, n, c), lambda i, s: (i, 0, 0)),
            pl.BlockSpec((1, c, hw), lambda i, s: (i, 0, 0)),
            pl.BlockSpec((1, n, c), lambda i, s: (i, 0, 0)),
        ],
        out_specs=pl.BlockSpec((1, 1, 128), lambda i, s: (i, 0, 0)),
    )
    part = pl.pallas_call(
        _loss_kernel,
        grid_spec=grid_spec,
        out_shape=jax.ShapeDtypeStruct((b, 1, 128), jnp.float32),
    )(homo_flat, kp1, w_kp1, kp1_desc, D, wdu)
    fos = jnp.sum(part[:, 0, 0]) / (b * n * _NUM_NEG)
    sos = jnp.sum(part[:, 0, 1]) / (b * n)
    return fos + sos
